# baseline (speedup 1.0000x reference)
import functools

import jax
import jax.numpy as jnp
from jax import lax
from jax.experimental import pallas as pl
from jax.experimental.pallas import tpu as pltpu
from jax.experimental.pallas import tpu_sc as plsc


_N_EMB = 1024
_DPAD = 128


def _vq_argmin_body(xt_ref, w_ref, idx_ref, wt_ref):
    xt = xt_ref[...]
    w = w_ref[...]
    M = xt.shape[0]
    scores = lax.dot_general(
        xt, w, (((1,), (0,)), ((), ())),
        preferred_element_type=jnp.float32,
        precision=lax.Precision.HIGHEST,
    )
    e2 = jnp.sum(w * w, axis=0, keepdims=True)
    dist = e2 - 2.0 * scores
    m = jnp.min(dist, axis=1, keepdims=True)
    ids = lax.broadcasted_iota(jnp.int32, (M, _N_EMB), 1)
    idx_ref[...] = jnp.min(jnp.where(dist == m, ids, jnp.int32(_N_EMB)),
                           axis=1, keepdims=True)
    if wt_ref is not None:
        D = w.shape[0]
        wt_ref[:, :D] = w.T
        wt_ref[:, D:] = jnp.zeros((_N_EMB, _DPAD - D), jnp.float32)


def _tc_argmin(xt, weight, emit_table):
    M = xt.shape[0]
    out_shape = [jax.ShapeDtypeStruct((M, 1), jnp.int32)]
    body = _vq_argmin_body
    if emit_table:
        out_shape.append(jax.ShapeDtypeStruct((_N_EMB, _DPAD), jnp.float32))
    else:
        body = functools.partial(_vq_argmin_body, wt_ref=None)
    return pl.pallas_call(body, out_shape=out_shape)(xt, weight)


def _sc_gather(table, idx):
    M = idx.shape[0]
    NC, NS = 2, 16
    b_per_w = M // (NC * NS)

    @functools.partial(
        pl.kernel,
        mesh=plsc.VectorSubcoreMesh(core_axis_name="c", subcore_axis_name="s"),
        out_type=jax.ShapeDtypeStruct((M, _DPAD), jnp.float32),
        scratch_types=[
            pltpu.VMEM((b_per_w,), jnp.int32),
            pltpu.VMEM((b_per_w, _DPAD), jnp.float32),
            pltpu.SemaphoreType.DMA,
        ],
    )
    def k(table_hbm, idx_hbm, out_hbm, idx_v, rows_v, sem):
        wid = lax.axis_index("s") * NC + lax.axis_index("c")
        base = wid * b_per_w
        pltpu.sync_copy(idx_hbm.at[pl.ds(base, b_per_w)], idx_v)
        pltpu.async_copy(table_hbm.at[idx_v], rows_v, sem).wait()
        pltpu.sync_copy(rows_v, out_hbm.at[pl.ds(base, b_per_w)])

    return k(table, idx)


def kernel(x, weight):
    B, D, H, W = x.shape
    P = H * W
    M = B * P
    Mh = M // 2
    Bh = B // 2
    xt = x.reshape(B, D, P).transpose(0, 2, 1).reshape(M, D)
    idx1, wt = _tc_argmin(xt[:Mh], weight, emit_table=True)
    idx2 = _tc_argmin(xt[Mh:], weight, emit_table=False)[0]
    rows1 = _sc_gather(wt, idx1.reshape(Mh))
    rows2 = _sc_gather(wt, idx2.reshape(Mh))
    res1 = rows1[:, :D].reshape(Bh, P, D).transpose(0, 2, 1)
    res2 = rows2[:, :D].reshape(Bh, P, D).transpose(0, 2, 1)
    res = jnp.concatenate([res1, res2], axis=0).reshape(B, D, H, W)
    idx = jnp.concatenate([idx1, idx2], axis=0).reshape(B, H, W)
    return res, idx


# speedup vs baseline: 1.2777x; 1.2777x over previous
import functools

import jax
import jax.numpy as jnp
from jax import lax
from jax.experimental import pallas as pl
from jax.experimental.pallas import tpu as pltpu
from jax.experimental.pallas import tpu_sc as plsc


_N_EMB = 1024
_DPAD = 128


def _vq_argmin_body(xt_ref, w_ref, idx_ref, wt_ref):
    xt = xt_ref[...]
    w = w_ref[...]
    M = xt.shape[0]
    scores = lax.dot_general(
        xt, w, (((1,), (0,)), ((), ())),
        preferred_element_type=jnp.float32,
        precision=lax.Precision.HIGHEST,
    )
    e2 = jnp.sum(w * w, axis=0, keepdims=True)
    dist = e2 - 2.0 * scores
    m = jnp.min(dist, axis=1, keepdims=True)
    ids = lax.broadcasted_iota(jnp.int32, (M, _N_EMB), 1)
    idx_ref[...] = jnp.min(jnp.where(dist == m, ids, jnp.int32(_N_EMB)),
                           axis=1, keepdims=True)
    D = w.shape[0]
    wt_ref[:, :D] = w.T


def _tc_argmin(xt, weight):
    M = xt.shape[0]
    return pl.pallas_call(
        _vq_argmin_body,
        out_shape=[
            jax.ShapeDtypeStruct((M, 1), jnp.int32),
            jax.ShapeDtypeStruct((_N_EMB, _DPAD), jnp.float32),
        ],
    )(xt, weight)


def _sc_gather(table, idx):
    M = idx.shape[0]
    NC, NS = 2, 16
    b_per_w = M // (NC * NS)

    @functools.partial(
        pl.kernel,
        mesh=plsc.VectorSubcoreMesh(core_axis_name="c", subcore_axis_name="s"),
        out_type=jax.ShapeDtypeStruct((M, _DPAD), jnp.float32),
        scratch_types=[
            pltpu.VMEM((b_per_w,), jnp.int32),
            pltpu.VMEM((b_per_w, _DPAD), jnp.float32),
            pltpu.SemaphoreType.DMA,
        ],
    )
    def k(table_hbm, idx_hbm, out_hbm, idx_v, rows_v, sem):
        wid = lax.axis_index("s") * NC + lax.axis_index("c")
        base = wid * b_per_w
        pltpu.sync_copy(idx_hbm.at[pl.ds(base, b_per_w)], idx_v)
        pltpu.async_copy(table_hbm.at[idx_v], rows_v, sem).wait()
        pltpu.sync_copy(rows_v, out_hbm.at[pl.ds(base, b_per_w)])

    return k(table, idx)


def kernel(x, weight):
    B, D, H, W = x.shape
    P = H * W
    M = B * P
    xt = x.reshape(B, D, P).transpose(0, 2, 1).reshape(M, D)
    idx, wt = _tc_argmin(xt, weight)
    rows = _sc_gather(wt, idx.reshape(M))
    res = rows[:, :D].reshape(B, P, D).transpose(0, 2, 1)
    return res.reshape(B, D, H, W), idx.reshape(B, H, W)
